# baseline (speedup 1.0000x reference)
import jax
import jax.numpy as jnp
from jax import lax
from jax.experimental import pallas as pl
from jax.experimental.pallas import tpu as pltpu
from jax.experimental.pallas import tpu_sc as plsc

B, S, H, V, P, T = 32, 512, 768, 30522, 512, 2
EPS = 1e-12
L = 16
NW = 32
SS = S // NW
NJ = H // L
NG = (B * S) // L
INV_H = 1.0 / H
MAGIC = 0x5F3759DF


def _rsqrt16(v):
    i = lax.bitcast_convert_type(v, jnp.int32)
    y = lax.bitcast_convert_type(MAGIC - (i >> 1), jnp.float32)
    for _ in range(3):
        y = y * (1.5 - 0.5 * v * y * y)
    return y


def _lanesum16(v):
    lane = lax.iota(jnp.int32, L)
    for sh in (8, 4, 2, 1):
        v = v + v.at[lane ^ sh].get(mode="promise_in_bounds")
    return v


def _emb_body(ids2, tt2, am_flat, word_hbm, pe_hbm, te2_hbm, gam_hbm, bet_hbm,
              out_hbm, mask_hbm,
              rowidx_v, rawids_v, rawtt_v, idsall_v, ttall_v, amall_v,
              maskall_v,
              b0, b1, b2, b3, o0, o1, ebuf_v, pe_v, te2_v, td_v, gam_v,
              bet_v, g0, g1, g2, g3, w0, w1, sst):
    wid = lax.axis_index("s") * 2 + lax.axis_index("c")
    base_s = wid * SS
    bufs = (b0, b1, b2, b3)
    obufs = (o0, o1)
    gsem = (g0, g1, g2, g3)
    wsem = (w0, w1)
    lane = lax.iota(jnp.int32, L)

    rowhi = wid // 8
    col = (wid % 8) * L
    rowidx_v[pl.ds(0, L)] = lane * 4 + rowhi
    rowidx_v[pl.ds(L, L)] = (lane + L) * 4 + rowhi
    pltpu.async_copy(ids2.at[rowidx_v], rawids_v, sst).wait()
    pltpu.async_copy(tt2.at[rowidx_v], rawtt_v, sst).wait()
    pltpu.sync_copy(am_flat.at[pl.ds(wid * S, S)], amall_v)
    pltpu.sync_copy(pe_hbm.at[pl.ds(base_s, SS)], pe_v)
    pltpu.sync_copy(te2_hbm, te2_v)
    pltpu.sync_copy(gam_hbm, gam_v)
    pltpu.sync_copy(bet_hbm, bet_v)

    def extract(k, c):
        idsall_v[k, pl.ds(0, L)] = rawids_v[k, pl.ds(col, L)]
        ttall_v[k, pl.ds(0, L)] = rawtt_v[k, pl.ds(col, L)]
        return c
    lax.fori_loop(0, B, extract, 0)

    def stage_tt(j, c):
        t0 = te2_v[0, pl.ds(j * L, L)]
        t1 = te2_v[1, pl.ds(j * L, L)]
        td_v[pl.ds(j * L, L)] = t1 - t0
        return c
    lax.fori_loop(0, NJ, stage_tt, 0)

    def stage_pe(i, c):
        def add_t0(j, c2):
            pe_v[i, pl.ds(j * L, L)] = (pe_v[i, pl.ds(j * L, L)]
                                        + te2_v[0, pl.ds(j * L, L)])
            return c2
        lax.fori_loop(0, NJ, add_t0, 0)
        return c
    lax.fori_loop(0, SS, stage_pe, 0)

    def mask_row(k, c):
        maskall_v[pl.ds(k * L, L)] = (1.0 - amall_v[pl.ds(k * L, L)]) * -10000.0
        return c
    lax.fori_loop(0, S // L, mask_row, 0)

    def compute_chunk(c, buf, obuf):
        ttw = ttall_v[c, pl.ds(0, L)].astype(jnp.float32)

        def row_body(i, c1):
            rbuf = buf.at[i]
            rpe = pe_v.at[i]
            robuf = obuf.at[i]
            ttf = _lanesum16(jnp.where(lane == i, ttw, 0.0))
            a1 = [jnp.zeros((L,), jnp.float32) for _ in range(4)]
            a2 = [jnp.zeros((L,), jnp.float32) for _ in range(4)]
            for j in range(NJ):
                e = (rbuf[pl.ds(j * L, L)] + rpe[pl.ds(j * L, L)]
                     + ttf * td_v[pl.ds(j * L, L)])
                ebuf_v[pl.ds(j * L, L)] = e
                a1[j % 4] = a1[j % 4] + e
                a2[j % 4] = a2[j % 4] + e * e
            mean_v = _lanesum16((a1[0] + a1[1]) + (a1[2] + a1[3])) * INV_H
            var_v = (_lanesum16((a2[0] + a2[1]) + (a2[2] + a2[3])) * INV_H
                     - mean_v * mean_v)
            rstd = _rsqrt16(var_v + EPS)
            for j in range(NJ):
                e = ebuf_v[pl.ds(j * L, L)]
                g = gam_v[pl.ds(j * L, L)]
                bt = bet_v[pl.ds(j * L, L)]
                robuf[pl.ds(j * L, L)] = (e - mean_v) * rstd * g + bt
            return c1
        lax.fori_loop(0, SS, row_body, 0)

    pltpu.async_copy(word_hbm.at[idsall_v.at[0]], b0, g0)
    pltpu.async_copy(word_hbm.at[idsall_v.at[1]], b1, g1)

    def k_body(k, c0):
        for r in range(4):
            c = k * 4 + r
            rr = (r + 2) % 4
            p = r % 2

            @pl.when(c + 2 < B)
            def _():
                pltpu.async_copy(word_hbm.at[idsall_v.at[c + 2]], bufs[rr],
                                 gsem[rr])

            pltpu.make_async_copy(word_hbm.at[pl.ds(0, SS)], bufs[r],
                                  gsem[r]).wait()

            @pl.when(c >= 2)
            def _():
                pltpu.make_async_copy(word_hbm.at[pl.ds(0, SS)], obufs[p],
                                      wsem[p]).wait()

            compute_chunk(c, bufs[r], obufs[p])
            tok = c * S + base_s
            pltpu.async_copy(obufs[p], out_hbm.at[pl.ds(tok, SS)], wsem[p])
        return c0
    lax.fori_loop(0, B // 4, k_body, 0)

    pltpu.make_async_copy(word_hbm.at[pl.ds(0, SS)], o0, w0).wait()
    pltpu.make_async_copy(word_hbm.at[pl.ds(0, SS)], o1, w1).wait()
    pltpu.sync_copy(maskall_v, mask_hbm.at[pl.ds(wid * S, S)])


@jax.jit
def _emb_call(ids2, tt2, am2, word, pe, te2, gamma, beta):
    mesh = plsc.VectorSubcoreMesh(core_axis_name="c", subcore_axis_name="s")
    k = pl.kernel(
        _emb_body, mesh=mesh,
        out_type=(jax.ShapeDtypeStruct((B * S, H), jnp.float32),
                  jax.ShapeDtypeStruct((B * S,), jnp.float32)),
        scratch_types=[
            pltpu.VMEM((NW,), jnp.int32),
            pltpu.VMEM((NW, 128), jnp.int32),
            pltpu.VMEM((NW, 128), jnp.int32),
            pltpu.VMEM((NW, L), jnp.int32),
            pltpu.VMEM((NW, L), jnp.int32),
            pltpu.VMEM((S,), jnp.float32),
            pltpu.VMEM((S,), jnp.float32),
            pltpu.VMEM((SS, H), jnp.float32),
            pltpu.VMEM((SS, H), jnp.float32),
            pltpu.VMEM((SS, H), jnp.float32),
            pltpu.VMEM((SS, H), jnp.float32),
            pltpu.VMEM((SS, H), jnp.float32),
            pltpu.VMEM((SS, H), jnp.float32),
            pltpu.VMEM((H,), jnp.float32),
            pltpu.VMEM((SS, H), jnp.float32),
            pltpu.VMEM((T, H), jnp.float32),
            pltpu.VMEM((H,), jnp.float32),
            pltpu.VMEM((H,), jnp.float32),
            pltpu.VMEM((H,), jnp.float32),
            pltpu.SemaphoreType.DMA,
            pltpu.SemaphoreType.DMA,
            pltpu.SemaphoreType.DMA,
            pltpu.SemaphoreType.DMA,
            pltpu.SemaphoreType.DMA,
            pltpu.SemaphoreType.DMA,
            pltpu.SemaphoreType.DMA,
        ],
    )
    return k(ids2, tt2, am2, word, pe, te2, gamma, beta)


def kernel(input_ids, attention_mask, token_type_ids, word_embeddings,
           position_embeddings, token_type_embeddings, ln_gamma, ln_beta):
    ids2 = input_ids.astype(jnp.int32).reshape(B * S // 128, 128)
    tt2 = token_type_ids.astype(jnp.int32).reshape(B * S // 128, 128)
    am_flat = attention_mask.astype(jnp.float32).reshape(B * S)
    out_flat, mask_flat = _emb_call(
        ids2, tt2, am_flat, word_embeddings, position_embeddings,
        token_type_embeddings, ln_gamma, ln_beta)
    return out_flat.reshape(B, S, H), mask_flat.reshape(B, S)


# speedup vs baseline: 3.1938x; 3.1938x over previous
import jax
import jax.numpy as jnp
from jax import lax
from jax.experimental import pallas as pl
from jax.experimental.pallas import tpu as pltpu
from jax.experimental.pallas import tpu_sc as plsc

B, S, H, V, P, T = 32, 512, 768, 30522, 512, 2
EPS = 1e-12
L = 16
NW = 32
SS = S // NW
RB = 256
MAGIC = 0x5F3759DF


def _sc_body(ids2, am_flat, word_hbm, out_hbm, mask_hbm,
             rowidx_v, rawids_v, idsall_v, amall_v, maskall_v,
             b0, b1, b2, b3, g0, g1, g2, g3, w0, w1, w2, w3, sst):
    wid = lax.axis_index("s") * 2 + lax.axis_index("c")
    base_s = wid * SS
    bufs = (b0, b1, b2, b3)
    gsem = (g0, g1, g2, g3)
    wsem = (w0, w1, w2, w3)
    lane = lax.iota(jnp.int32, L)

    rowhi = wid // 8
    col = (wid % 8) * L
    rowidx_v[pl.ds(0, L)] = lane * 4 + rowhi
    rowidx_v[pl.ds(L, L)] = (lane + L) * 4 + rowhi
    pltpu.async_copy(ids2.at[rowidx_v], rawids_v, sst).wait()
    pltpu.sync_copy(am_flat.at[pl.ds(wid * S, S)], amall_v)

    def extract(k, c):
        idsall_v[k, pl.ds(0, L)] = rawids_v[k, pl.ds(col, L)]
        return c
    lax.fori_loop(0, B, extract, 0)

    def mask_row(k, c):
        maskall_v[pl.ds(k * L, L)] = (1.0 - amall_v[pl.ds(k * L, L)]) * -10000.0
        return c
    lax.fori_loop(0, S // L, mask_row, 0)

    pltpu.async_copy(word_hbm.at[idsall_v.at[0]], b0, g0)
    pltpu.async_copy(word_hbm.at[idsall_v.at[1]], b1, g1)

    def k_body(k, c0):
        for r in range(4):
            c = k * 4 + r
            rr = (r + 2) % 4

            @pl.when(c >= 2)
            def _():
                pltpu.make_async_copy(word_hbm.at[pl.ds(0, SS)], bufs[rr],
                                      wsem[rr]).wait()

            @pl.when(c + 2 < B)
            def _():
                pltpu.async_copy(word_hbm.at[idsall_v.at[c + 2]], bufs[rr],
                                 gsem[rr])

            pltpu.make_async_copy(word_hbm.at[pl.ds(0, SS)], bufs[r],
                                  gsem[r]).wait()
            tok = c * S + base_s
            pltpu.async_copy(bufs[r], out_hbm.at[pl.ds(tok, SS)], wsem[r])
        return c0
    lax.fori_loop(0, B // 4, k_body, 0)

    pltpu.make_async_copy(word_hbm.at[pl.ds(0, SS)], b2, w2).wait()
    pltpu.make_async_copy(word_hbm.at[pl.ds(0, SS)], b3, w3).wait()
    pltpu.sync_copy(maskall_v, mask_hbm.at[pl.ds(wid * S, S)])


@jax.jit
def _sc_gather(ids2, am_flat, word):
    mesh = plsc.VectorSubcoreMesh(core_axis_name="c", subcore_axis_name="s")
    k = pl.kernel(
        _sc_body, mesh=mesh,
        out_type=(jax.ShapeDtypeStruct((B * S, H), jnp.float32),
                  jax.ShapeDtypeStruct((B * S,), jnp.float32)),
        scratch_types=[
            pltpu.VMEM((NW,), jnp.int32),
            pltpu.VMEM((NW, 128), jnp.int32),
            pltpu.VMEM((NW, L), jnp.int32),
            pltpu.VMEM((S,), jnp.float32),
            pltpu.VMEM((S,), jnp.float32),
            pltpu.VMEM((SS, H), jnp.float32),
            pltpu.VMEM((SS, H), jnp.float32),
            pltpu.VMEM((SS, H), jnp.float32),
            pltpu.VMEM((SS, H), jnp.float32),
            pltpu.SemaphoreType.DMA,
            pltpu.SemaphoreType.DMA,
            pltpu.SemaphoreType.DMA,
            pltpu.SemaphoreType.DMA,
            pltpu.SemaphoreType.DMA,
            pltpu.SemaphoreType.DMA,
            pltpu.SemaphoreType.DMA,
            pltpu.SemaphoreType.DMA,
            pltpu.SemaphoreType.DMA,
        ],
    )
    return k(ids2, am_flat, word)


def _ln_body(we_ref, pe_ref, ttf_ref, te2_ref, gam_ref, bet_ref, out_ref):
    e = (we_ref[...] + pe_ref[...] + te2_ref[0:1, :]
         + ttf_ref[...] * (te2_ref[1:2, :] - te2_ref[0:1, :]))
    mean = jnp.mean(e, axis=1, keepdims=True)
    var = jnp.mean(jnp.square(e - mean), axis=1, keepdims=True)
    out_ref[...] = ((e - mean) * lax.rsqrt(var + EPS) * gam_ref[...]
                    + bet_ref[...])


@jax.jit
def _tc_ln(we_flat, pe, ttf, te2, gamma, beta):
    grid = (S // RB, B)
    return pl.pallas_call(
        _ln_body,
        grid=grid,
        in_specs=[
            pl.BlockSpec((RB, H), lambda pc, b: (b * (S // RB) + pc, 0)),
            pl.BlockSpec((RB, H), lambda pc, b: (pc, 0)),
            pl.BlockSpec((RB, 1), lambda pc, b: (b * (S // RB) + pc, 0)),
            pl.BlockSpec((T, H), lambda pc, b: (0, 0)),
            pl.BlockSpec((1, H), lambda pc, b: (0, 0)),
            pl.BlockSpec((1, H), lambda pc, b: (0, 0)),
        ],
        out_specs=pl.BlockSpec((RB, H), lambda pc, b: (b * (S // RB) + pc, 0)),
        out_shape=jax.ShapeDtypeStruct((B * S, H), jnp.float32),
    )(we_flat, pe, ttf, te2, gamma, beta)


def kernel(input_ids, attention_mask, token_type_ids, word_embeddings,
           position_embeddings, token_type_embeddings, ln_gamma, ln_beta):
    ids2 = input_ids.astype(jnp.int32).reshape(B * S // 128, 128)
    am_flat = attention_mask.astype(jnp.float32).reshape(B * S)
    we_flat, mask_flat = _sc_gather(ids2, am_flat, word_embeddings)
    ttf = token_type_ids.astype(jnp.float32).reshape(B * S, 1)
    out_flat = _tc_ln(we_flat, position_embeddings, ttf,
                      token_type_embeddings, ln_gamma.reshape(1, H),
                      ln_beta.reshape(1, H))
    return out_flat.reshape(B, S, H), mask_flat.reshape(B, S)


# speedup vs baseline: 3.7570x; 1.1763x over previous
import jax
import jax.numpy as jnp
from jax import lax
from jax.experimental import pallas as pl
from jax.experimental.pallas import tpu as pltpu
from jax.experimental.pallas import tpu_sc as plsc

B, S, H, V, P, T = 32, 512, 768, 30522, 512, 2
EPS = 1e-12
L = 16
NW = 32
SS = S // NW
RB = 512
MAGIC = 0x5F3759DF


def _sc_body(ids2, am_flat, word_hbm, out_hbm, mask_hbm,
             rowidx_v, rawids_v, idsall_v, amall_v, maskall_v,
             b0, b1, b2, b3, g0, g1, g2, g3, w0, w1, w2, w3, sst):
    wid = lax.axis_index("s") * 2 + lax.axis_index("c")
    base_s = wid * SS
    bufs = (b0, b1, b2, b3)
    gsem = (g0, g1, g2, g3)
    wsem = (w0, w1, w2, w3)
    lane = lax.iota(jnp.int32, L)

    rowhi = wid // 8
    col = (wid % 8) * L
    rowidx_v[pl.ds(0, L)] = lane * 4 + rowhi
    rowidx_v[pl.ds(L, L)] = (lane + L) * 4 + rowhi
    pltpu.async_copy(ids2.at[rowidx_v], rawids_v, sst).wait()
    pltpu.sync_copy(am_flat.at[pl.ds(wid * S, S)], amall_v)

    def extract(k, c):
        idsall_v[k, pl.ds(0, L)] = rawids_v[k, pl.ds(col, L)]
        return c
    lax.fori_loop(0, B, extract, 0)

    def mask_row(k, c):
        maskall_v[pl.ds(k * L, L)] = (1.0 - amall_v[pl.ds(k * L, L)]) * -10000.0
        return c
    lax.fori_loop(0, S // L, mask_row, 0)

    pltpu.async_copy(word_hbm.at[idsall_v.at[0]], b0, g0)
    pltpu.async_copy(word_hbm.at[idsall_v.at[1]], b1, g1)

    def k_body(k, c0):
        for r in range(4):
            c = k * 4 + r
            rr = (r + 2) % 4

            @pl.when(c >= 2)
            def _():
                pltpu.make_async_copy(word_hbm.at[pl.ds(0, SS)], bufs[rr],
                                      wsem[rr]).wait()

            @pl.when(c + 2 < B)
            def _():
                pltpu.async_copy(word_hbm.at[idsall_v.at[c + 2]], bufs[rr],
                                 gsem[rr])

            pltpu.make_async_copy(word_hbm.at[pl.ds(0, SS)], bufs[r],
                                  gsem[r]).wait()
            tok = c * S + base_s
            pltpu.async_copy(bufs[r], out_hbm.at[pl.ds(tok, SS)], wsem[r])
        return c0
    lax.fori_loop(0, B // 4, k_body, 0)

    pltpu.make_async_copy(word_hbm.at[pl.ds(0, SS)], b2, w2).wait()
    pltpu.make_async_copy(word_hbm.at[pl.ds(0, SS)], b3, w3).wait()
    pltpu.sync_copy(maskall_v, mask_hbm.at[pl.ds(wid * S, S)])


@jax.jit
def _sc_gather(ids2, am_flat, word):
    mesh = plsc.VectorSubcoreMesh(core_axis_name="c", subcore_axis_name="s")
    k = pl.kernel(
        _sc_body, mesh=mesh,
        out_type=(jax.ShapeDtypeStruct((B * S, H), jnp.float32),
                  jax.ShapeDtypeStruct((B * S,), jnp.float32)),
        scratch_types=[
            pltpu.VMEM((NW,), jnp.int32),
            pltpu.VMEM((NW, 128), jnp.int32),
            pltpu.VMEM((NW, L), jnp.int32),
            pltpu.VMEM((S,), jnp.float32),
            pltpu.VMEM((S,), jnp.float32),
            pltpu.VMEM((SS, H), jnp.float32),
            pltpu.VMEM((SS, H), jnp.float32),
            pltpu.VMEM((SS, H), jnp.float32),
            pltpu.VMEM((SS, H), jnp.float32),
            pltpu.SemaphoreType.DMA,
            pltpu.SemaphoreType.DMA,
            pltpu.SemaphoreType.DMA,
            pltpu.SemaphoreType.DMA,
            pltpu.SemaphoreType.DMA,
            pltpu.SemaphoreType.DMA,
            pltpu.SemaphoreType.DMA,
            pltpu.SemaphoreType.DMA,
            pltpu.SemaphoreType.DMA,
        ],
    )
    return k(ids2, am_flat, word)


def _ln_body(we_ref, pe_ref, ttf_ref, te2_ref, gam_ref, bet_ref, out_ref):
    e = (we_ref[...] + pe_ref[...] + te2_ref[0:1, :]
         + ttf_ref[...] * (te2_ref[1:2, :] - te2_ref[0:1, :]))
    mean = jnp.mean(e, axis=1, keepdims=True)
    var = jnp.mean(jnp.square(e - mean), axis=1, keepdims=True)
    out_ref[...] = ((e - mean) * lax.rsqrt(var + EPS) * gam_ref[...]
                    + bet_ref[...])


@jax.jit
def _tc_ln(we_flat, pe, ttf, te2, gamma, beta):
    grid = (S // RB, B)
    return pl.pallas_call(
        _ln_body,
        grid=grid,
        in_specs=[
            pl.BlockSpec((RB, H), lambda pc, b: (b * (S // RB) + pc, 0)),
            pl.BlockSpec((RB, H), lambda pc, b: (pc, 0)),
            pl.BlockSpec((RB, 1), lambda pc, b: (b * (S // RB) + pc, 0)),
            pl.BlockSpec((T, H), lambda pc, b: (0, 0)),
            pl.BlockSpec((1, H), lambda pc, b: (0, 0)),
            pl.BlockSpec((1, H), lambda pc, b: (0, 0)),
        ],
        out_specs=pl.BlockSpec((RB, H), lambda pc, b: (b * (S // RB) + pc, 0)),
        out_shape=jax.ShapeDtypeStruct((B * S, H), jnp.float32),
    )(we_flat, pe, ttf, te2, gamma, beta)


def kernel(input_ids, attention_mask, token_type_ids, word_embeddings,
           position_embeddings, token_type_embeddings, ln_gamma, ln_beta):
    ids2 = input_ids.astype(jnp.int32).reshape(B * S // 128, 128)
    am_flat = attention_mask.astype(jnp.float32).reshape(B * S)
    we_flat, mask_flat = _sc_gather(ids2, am_flat, word_embeddings)
    ttf = token_type_ids.astype(jnp.float32).reshape(B * S, 1)
    out_flat = _tc_ln(we_flat, position_embeddings, ttf,
                      token_type_embeddings, ln_gamma.reshape(1, H),
                      ln_beta.reshape(1, H))
    return out_flat.reshape(B, S, H), mask_flat.reshape(B, S)


# speedup vs baseline: 3.7638x; 1.0018x over previous
import jax
import jax.numpy as jnp
from jax import lax
from jax.experimental import pallas as pl
from jax.experimental.pallas import tpu as pltpu
from jax.experimental.pallas import tpu_sc as plsc

B, S, H, V, P, T = 32, 512, 768, 30522, 512, 2
EPS = 1e-12
L = 16
NW = 32
SS = S // NW
RB = 512
MAGIC = 0x5F3759DF


def _sc_body(ids2, am_flat, word_hbm, out_hbm, mask_hbm,
             rowidx_v, rawids_v, idsall_v, amall_v, maskall_v,
             b0, b1, b2, b3, g0, g1, g2, g3, w0, w1, w2, w3, sst):
    wid = lax.axis_index("s") * 2 + lax.axis_index("c")
    base_s = wid * SS
    bufs = (b0, b1, b2, b3)
    gsem = (g0, g1, g2, g3)
    wsem = (w0, w1, w2, w3)
    lane = lax.iota(jnp.int32, L)

    rowhi = wid // 8
    col = (wid % 8) * L
    rowidx_v[pl.ds(0, L)] = lane * 4 + rowhi
    rowidx_v[pl.ds(L, L)] = (lane + L) * 4 + rowhi
    pltpu.async_copy(ids2.at[rowidx_v], rawids_v, sst).wait()
    pltpu.sync_copy(am_flat.at[pl.ds(wid * S, S)], amall_v)

    def extract(k, c):
        idsall_v[k // 2, pl.ds((k % 2) * L, L)] = rawids_v[k, pl.ds(col, L)]
        return c
    lax.fori_loop(0, B, extract, 0)

    def mask_row(k, c):
        maskall_v[pl.ds(k * L, L)] = (1.0 - amall_v[pl.ds(k * L, L)]) * -10000.0
        return c
    lax.fori_loop(0, S // L, mask_row, 0)

    pltpu.async_copy(word_hbm.at[idsall_v.at[0]], b0, g0)
    pltpu.async_copy(word_hbm.at[idsall_v.at[1]], b1, g1)

    NC = B // 2

    def k_body(k, c0):
        for r in range(4):
            c = k * 4 + r
            rr = (r + 2) % 4

            @pl.when(c >= 2)
            def _():
                pltpu.make_async_copy(word_hbm.at[pl.ds(0, 2 * SS)],
                                      bufs[rr], wsem[rr]).wait()

            @pl.when(c + 2 < NC)
            def _():
                pltpu.async_copy(word_hbm.at[idsall_v.at[c + 2]], bufs[rr],
                                 gsem[rr])

            pltpu.make_async_copy(word_hbm.at[pl.ds(0, 2 * SS)], bufs[r],
                                  gsem[r]).wait()
            tok0 = (2 * c) * S + base_s
            tok1 = (2 * c + 1) * S + base_s
            pltpu.async_copy(bufs[r].at[pl.ds(0, SS)],
                             out_hbm.at[pl.ds(tok0, SS)], wsem[r])
            pltpu.async_copy(bufs[r].at[pl.ds(SS, SS)],
                             out_hbm.at[pl.ds(tok1, SS)], wsem[r])
        return c0
    lax.fori_loop(0, NC // 4, k_body, 0)

    pltpu.make_async_copy(word_hbm.at[pl.ds(0, 2 * SS)], b2, w2).wait()
    pltpu.make_async_copy(word_hbm.at[pl.ds(0, 2 * SS)], b3, w3).wait()
    pltpu.sync_copy(maskall_v, mask_hbm.at[pl.ds(wid * S, S)])


@jax.jit
def _sc_gather(ids2, am_flat, word):
    mesh = plsc.VectorSubcoreMesh(core_axis_name="c", subcore_axis_name="s")
    k = pl.kernel(
        _sc_body, mesh=mesh,
        out_type=(jax.ShapeDtypeStruct((B * S, H), jnp.float32),
                  jax.ShapeDtypeStruct((B * S,), jnp.float32)),
        scratch_types=[
            pltpu.VMEM((NW,), jnp.int32),
            pltpu.VMEM((NW, 128), jnp.int32),
            pltpu.VMEM((NW // 2, 2 * L), jnp.int32),
            pltpu.VMEM((S,), jnp.float32),
            pltpu.VMEM((S,), jnp.float32),
            pltpu.VMEM((2 * SS, H), jnp.float32),
            pltpu.VMEM((2 * SS, H), jnp.float32),
            pltpu.VMEM((2 * SS, H), jnp.float32),
            pltpu.VMEM((2 * SS, H), jnp.float32),
            pltpu.SemaphoreType.DMA,
            pltpu.SemaphoreType.DMA,
            pltpu.SemaphoreType.DMA,
            pltpu.SemaphoreType.DMA,
            pltpu.SemaphoreType.DMA,
            pltpu.SemaphoreType.DMA,
            pltpu.SemaphoreType.DMA,
            pltpu.SemaphoreType.DMA,
            pltpu.SemaphoreType.DMA,
        ],
    )
    return k(ids2, am_flat, word)


def _ln_body(we_ref, pe_ref, ttf_ref, te2_ref, gam_ref, bet_ref, out_ref):
    e = (we_ref[...] + pe_ref[...] + te2_ref[0:1, :]
         + ttf_ref[...] * (te2_ref[1:2, :] - te2_ref[0:1, :]))
    mean = jnp.mean(e, axis=1, keepdims=True)
    var = jnp.mean(jnp.square(e - mean), axis=1, keepdims=True)
    out_ref[...] = ((e - mean) * lax.rsqrt(var + EPS) * gam_ref[...]
                    + bet_ref[...])


@jax.jit
def _tc_ln(we_flat, pe, ttf, te2, gamma, beta):
    grid = (S // RB, B)
    return pl.pallas_call(
        _ln_body,
        grid=grid,
        in_specs=[
            pl.BlockSpec((RB, H), lambda pc, b: (b * (S // RB) + pc, 0)),
            pl.BlockSpec((RB, H), lambda pc, b: (pc, 0)),
            pl.BlockSpec((RB, 1), lambda pc, b: (b * (S // RB) + pc, 0)),
            pl.BlockSpec((T, H), lambda pc, b: (0, 0)),
            pl.BlockSpec((1, H), lambda pc, b: (0, 0)),
            pl.BlockSpec((1, H), lambda pc, b: (0, 0)),
        ],
        out_specs=pl.BlockSpec((RB, H), lambda pc, b: (b * (S // RB) + pc, 0)),
        out_shape=jax.ShapeDtypeStruct((B * S, H), jnp.float32),
    )(we_flat, pe, ttf, te2, gamma, beta)


def kernel(input_ids, attention_mask, token_type_ids, word_embeddings,
           position_embeddings, token_type_embeddings, ln_gamma, ln_beta):
    ids2 = input_ids.astype(jnp.int32).reshape(B * S // 128, 128)
    am_flat = attention_mask.astype(jnp.float32).reshape(B * S)
    we_flat, mask_flat = _sc_gather(ids2, am_flat, word_embeddings)
    ttf = token_type_ids.astype(jnp.float32).reshape(B * S, 1)
    out_flat = _tc_ln(we_flat, position_embeddings, ttf,
                      token_type_embeddings, ln_gamma.reshape(1, H),
                      ln_beta.reshape(1, H))
    return out_flat.reshape(B, S, H), mask_flat.reshape(B, S)
